# Initial kernel scaffold; baseline (speedup 1.0000x reference)
import jax
import jax.numpy as jnp
from jax.experimental import pallas as pl


def kernel(x, gate_w, w1, w2, w3):
    raise NotImplementedError("write your pallas kernel here")


# speedup vs baseline: 6.4354x; 6.4354x over previous
import jax
import jax.numpy as jnp
from jax import lax
from jax.experimental import pallas as pl
from jax.experimental.pallas import tpu as pltpu

DIM = 768
NUM_EXPERTS = 8
INTER = 2048
TOP_K = 2
T = 16


def _routing_combine(x, gate_w):
    scores = lax.dot_general(
        x, gate_w, (((1,), (1,)), ((), ())),
        preferred_element_type=jnp.float32)
    m = jnp.max(scores, axis=-1, keepdims=True)
    p = jnp.exp(scores - m)
    p = p / jnp.sum(p, axis=-1, keepdims=True)
    eidx = lax.broadcasted_iota(jnp.int32, (T, NUM_EXPERTS), 1)
    m1 = jnp.max(p, axis=-1, keepdims=True)
    i1 = jnp.min(jnp.where(p == m1, eidx, NUM_EXPERTS), axis=-1, keepdims=True)
    oh1 = eidx == i1
    p_rest = jnp.where(oh1, -1.0, p)
    m2 = jnp.max(p_rest, axis=-1, keepdims=True)
    i2 = jnp.min(jnp.where(p_rest == m2, eidx, NUM_EXPERTS),
                 axis=-1, keepdims=True)
    oh2 = eidx == i2
    c = jnp.where(oh1 | oh2, p, 0.0)
    return c / jnp.sum(c, axis=-1, keepdims=True)


def _moe_body(x_ref, gate_ref, w1_ref, w2_ref, w3_ref, out_ref, c_ref):
    e = pl.program_id(0)

    @pl.when(e == 0)
    def _init():
        c_ref[...] = _routing_combine(x_ref[...], gate_ref[...])
        out_ref[...] = jnp.zeros_like(out_ref)

    xv = x_ref[...]
    w1e = w1_ref[0]
    w3e = w3_ref[0]
    w2e = w2_ref[0]
    dn = (((1,), (1,)), ((), ()))
    h1 = lax.dot_general(xv, w1e, dn, preferred_element_type=jnp.float32)
    h3 = lax.dot_general(xv, w3e, dn, preferred_element_type=jnp.float32)
    h = h1 * lax.logistic(h1) * h3
    oute = lax.dot_general(h, w2e, dn, preferred_element_type=jnp.float32)
    eidx = lax.broadcasted_iota(jnp.int32, (T, NUM_EXPERTS), 1)
    col = jnp.sum(jnp.where(eidx == e, c_ref[...], 0.0),
                  axis=-1, keepdims=True)
    out_ref[...] += col * oute


def kernel(x, gate_w, w1, w2, w3):
    original_shape = x.shape
    xf = x.reshape(-1, DIM)
    out = pl.pallas_call(
        _moe_body,
        grid=(NUM_EXPERTS,),
        in_specs=[
            pl.BlockSpec((T, DIM), lambda e: (0, 0)),
            pl.BlockSpec((NUM_EXPERTS, DIM), lambda e: (0, 0)),
            pl.BlockSpec((1, INTER, DIM), lambda e: (e, 0, 0)),
            pl.BlockSpec((1, DIM, INTER), lambda e: (e, 0, 0)),
            pl.BlockSpec((1, INTER, DIM), lambda e: (e, 0, 0)),
        ],
        out_specs=pl.BlockSpec((T, DIM), lambda e: (0, 0)),
        out_shape=jax.ShapeDtypeStruct((T, DIM), jnp.float32),
        scratch_shapes=[pltpu.VMEM((T, NUM_EXPERTS), jnp.float32)],
    )(xf, gate_w, w1, w2, w3)
    return out.reshape(original_shape)
